# baseline (speedup 1.0000x reference)
import functools

import jax
import jax.numpy as jnp
from jax import lax
from jax.experimental import pallas as pl
from jax.experimental.pallas import tpu as pltpu
from jax.experimental.pallas import tpu_sc as plsc

NT = 32768
NE = 64
NC = 2
NS = 16
NW = NC * NS
TPW = NT // NW
CHUNK = 256
NCHUNKS = TPW // CHUNK
L = 16
NV = NE // L
WPT = NE * L // NS

_mesh = plsc.VectorSubcoreMesh(core_axis_name="c", subcore_axis_name="s")


@functools.partial(
    pl.kernel,
    out_type=[
        jax.ShapeDtypeStruct((NT,), jnp.int32),
        jax.ShapeDtypeStruct((NT,), jnp.float32),
        jax.ShapeDtypeStruct((NC * NE,), jnp.int32),
        jax.ShapeDtypeStruct((NC * NE * L,), jnp.float32),
    ],
    mesh=_mesh,
    compiler_params=pltpu.CompilerParams(needs_layout_passes=False),
    scratch_types=[
        pltpu.VMEM((CHUNK * NE,), jnp.float32),
        pltpu.VMEM((CHUNK * NE,), jnp.float32),
        pltpu.VMEM((NE * L,), jnp.float32),
        pltpu.VMEM((NE * L,), jnp.float32),
        pltpu.VMEM((TPW,), jnp.int32),
        pltpu.VMEM((TPW,), jnp.float32),
        pltpu.VMEM((NE,), jnp.int32),
        pltpu.VMEM((NS * WPT,), jnp.float32),
        pltpu.VMEM((NS * NE,), jnp.int32),
        pltpu.VMEM((NE,), jnp.float32),
        pltpu.VMEM_SHARED((NS * NE * L,), jnp.float32),
        pltpu.VMEM_SHARED((NS * NE,), jnp.int32),
        pltpu.SemaphoreType.DMA,
        pltpu.SemaphoreType.DMA,
    ],
)
def _gate_kernel(x_hbm, sample_hbm, mult_hbm, cnt_hbm, psum_hbm,
                 buf0, buf1, tbuf, psum_t, sample_buf, mult_buf, cnt_buf,
                 agg_ps, agg_ct, stage, sh_ps, sh_ct, sem0, sem1):
    cid = lax.axis_index("c")
    sid = lax.axis_index("s")
    wid = cid * NS + sid
    tok0 = wid * TPW

    idx0 = lax.iota(jnp.int32, L)
    ones_i = jnp.ones((L,), jnp.int32)
    z16f = jnp.zeros((L,), jnp.float32)
    z16i = jnp.zeros((L,), jnp.int32)
    bufs = (buf0, buf1)
    sems = (sem0, sem1)

    cps = [None] * NCHUNKS
    cps[0] = pltpu.async_copy(
        x_hbm.at[pl.ds(tok0 * NE, CHUNK * NE)], bufs[0], sems[0])

    for e in range(NE):
        psum_t[pl.ds(e * L, L)] = z16f
    for j in range(NV):
        cnt_buf[pl.ds(j * L, L)] = z16i

    for k in range(NCHUNKS):
        b = k % 2
        if k + 1 < NCHUNKS:
            cps[k + 1] = pltpu.async_copy(
                x_hbm.at[pl.ds((tok0 + (k + 1) * CHUNK) * NE, CHUNK * NE)],
                bufs[1 - b], sems[1 - b])
        cps[k].wait()
        buf = bufs[b]

        def gbody(g, _, k=k, buf=buf):
            gidx = g * (L * NE) + idx0 * NE
            m = plsc.load_gather(buf, [gidx])
            amax = z16i
            tbuf[pl.ds(0, L)] = m
            for e in range(1, NE):
                v = plsc.load_gather(buf, [gidx + e])
                amax = jnp.where(v > m, jnp.int32(e), amax)
                m = jnp.maximum(m, v)
                tbuf[pl.ds(e * L, L)] = v
            s = z16f
            for e in range(NE):
                ex = jnp.exp(tbuf[pl.ds(e * L, L)] - m)
                s = s + ex
                tbuf[pl.ds(e * L, L)] = ex
            r = 1.0 / s
            off = k * CHUNK + g * L
            sample_buf[pl.ds(off, L)] = amax
            mult_buf[pl.ds(off, L)] = r
            plsc.addupdate_scatter(cnt_buf, [amax], ones_i)
            for e in range(NE):
                p = tbuf[pl.ds(e * L, L)] * r
                plsc.addupdate(psum_t.at[pl.ds(e * L, L)], p)
            return 0

        lax.fori_loop(0, CHUNK // L, gbody, 0)

    pltpu.sync_copy(sample_buf, sample_hbm.at[pl.ds(tok0, TPW)])
    pltpu.sync_copy(mult_buf, mult_hbm.at[pl.ds(tok0, TPW)])

    pltpu.sync_copy(psum_t, sh_ps.at[pl.ds(sid * NE * L, NE * L)])
    pltpu.sync_copy(cnt_buf, sh_ct.at[pl.ds(sid * NE, NE)])
    plsc.subcore_barrier()

    for rr in range(NS):
        pltpu.sync_copy(sh_ps.at[pl.ds(rr * NE * L + sid * WPT, WPT)],
                        agg_ps.at[pl.ds(rr * WPT, WPT)])
    accp = [z16f for _ in range(WPT // L)]
    for rr in range(NS):
        for j in range(WPT // L):
            accp[j] = accp[j] + agg_ps[pl.ds(rr * WPT + j * L, L)]
    for j in range(WPT // L):
        stage[pl.ds(j * L, L)] = accp[j]
    pltpu.sync_copy(stage.at[pl.ds(0, WPT)],
                    psum_hbm.at[pl.ds(cid * NE * L + sid * WPT, WPT)])

    @pl.when(sid == 0)
    def _():
        pltpu.sync_copy(sh_ct, agg_ct)
        accc = [z16i for _ in range(NV)]
        for rr in range(NS):
            for j in range(NV):
                accc[j] = accc[j] + agg_ct[pl.ds(rr * NE + j * L, L)]
        for j in range(NV):
            cnt_buf[pl.ds(j * L, L)] = accc[j]
        pltpu.sync_copy(cnt_buf, cnt_hbm.at[pl.ds(cid * NE, NE)])


def _loss_body(cnt_ref, ps_ref, out_ref):
    cntf = cnt_ref[...].astype(jnp.float32)
    ps = ps_ref[...]
    f2 = jnp.sum(cntf, axis=0, keepdims=True) * (1.0 / NT)
    pm2 = jnp.sum(ps, axis=0, keepdims=True) * (1.0 / NT)
    out_ref[...] = jnp.float32(NE) * jnp.sum(pm2 * f2, axis=1, keepdims=True)


def kernel(logits):
    x = logits.reshape(-1)
    sample, mult, cnt, psflat = _gate_kernel(x)
    ps = psflat.reshape(NC, NE, L).transpose(0, 2, 1).reshape(NC * L, NE)
    loss = pl.pallas_call(
        _loss_body,
        out_shape=jax.ShapeDtypeStruct((1, 1), jnp.float32),
    )(cnt.reshape(NC, NE), ps)
    return sample, mult.reshape(NT, 1), loss.reshape(())
